# baseline (speedup 1.0000x reference)
import jax
import jax.numpy as jnp
from jax.experimental import pallas as pl
from jax.experimental.pallas import tpu as pltpu

B, N, F, CH, H = 1024, 200, 128, 32, 512
BT = 64
NT = B // BT
ZG = 4
NG = 8


def _body(x_ref, a_ref, wcb_ref, bch_ref, w1_ref, b1_ref, w2_ref, b2_ref,
          w3_ref, b3_ref, w4_ref, b4_ref, out_ref, y_scr, z_scr, h_scr):
    f32 = jnp.float32
    xr = x_ref[...].reshape(BT * N, F)
    y_scr[...] = jnp.dot(xr, wcb_ref[...], preferred_element_type=f32)
    a = a_ref[...]

    def zstep(g, carry):
        base = g * (ZG * N)
        rhs = jnp.concatenate(
            [y_scr[pl.ds(base + k * N, N), CH:2 * CH] for k in range(ZG)],
            axis=1)
        zz = jnp.dot(a, rhs, preferred_element_type=f32)
        for k in range(ZG):
            z_scr[pl.ds(base + k * N, N), :] = zz[:, k * CH:(k + 1) * CH]
        return carry

    jax.lax.fori_loop(0, BT // ZG, zstep, 0)

    h = y_scr[:, :CH] + z_scr[...] + bch_ref[...]
    h = jnp.where(h > 0, h, jnp.exp(jnp.minimum(h, 0.0)) - 1.0)
    h_scr[...] = h.reshape(BT, N, CH)

    def d1step(g, acc):
        blk = h_scr[:, pl.ds(g * NG, NG), :].reshape(BT, NG * CH)
        w1blk = w1_ref[pl.ds(g * NG * CH, NG * CH), :]
        return acc + jnp.dot(blk, w1blk, preferred_element_type=f32)

    acc = jax.lax.fori_loop(0, N // NG, d1step,
                            jnp.zeros((BT, H), f32))
    o1 = jnp.maximum(acc + b1_ref[...], 0.0)
    o2 = jnp.maximum(jnp.dot(o1, w2_ref[...], preferred_element_type=f32)
                     + b2_ref[...], 0.0)
    o3 = jnp.maximum(jnp.dot(o2, w3_ref[...], preferred_element_type=f32)
                     + b3_ref[...], 0.0)
    o4 = jnp.dot(o3, w4_ref[...], preferred_element_type=f32) + b4_ref[...]
    out_ref[...] = jax.nn.sigmoid(o4)


def kernel(x, a, W_cheb, b_cheb, W1, b1, W2, b2, W3, b3, W4, b4):
    wcb = jnp.concatenate([W_cheb[0], W_cheb[1]], axis=1)
    bch = b_cheb.reshape(1, CH)
    b1r, b2r, b3r, b4r = (b1.reshape(1, -1), b2.reshape(1, -1),
                          b3.reshape(1, -1), b4.reshape(1, -1))
    full = lambda shape: pl.BlockSpec(shape, lambda i: (0,) * len(shape))
    return pl.pallas_call(
        _body,
        grid=(NT,),
        in_specs=[
            pl.BlockSpec((BT, N, F), lambda i: (i, 0, 0)),
            full((N, N)),
            full((F, 2 * CH)),
            full((1, CH)),
            full((N * CH, H)),
            full((1, H)),
            full((H, H // 2)),
            full((1, H // 2)),
            full((H // 2, H // 4)),
            full((1, H // 4)),
            full((H // 4, 1)),
            full((1, 1)),
        ],
        out_specs=pl.BlockSpec((BT, 1), lambda i: (i, 0)),
        out_shape=jax.ShapeDtypeStruct((B, 1), jnp.float32),
        scratch_shapes=[
            pltpu.VMEM((BT * N, 2 * CH), jnp.float32),
            pltpu.VMEM((BT * N, CH), jnp.float32),
            pltpu.VMEM((BT, N, CH), jnp.float32),
        ],
    )(x, a, wcb, bch, W1, b1r, W2, b2r, W3, b3r, W4, b4r)


# speedup vs baseline: 1.0758x; 1.0758x over previous
import jax
import jax.numpy as jnp
from jax.experimental import pallas as pl
from jax.experimental.pallas import tpu as pltpu

B, N, F, CH, H = 1024, 200, 128, 32, 512
BT = 128
NT = B // BT
ZG = 4
NG = 8


def _body(x_ref, a_ref, wcb_ref, bch_ref, w1_ref, b1_ref, w2_ref, b2_ref,
          w3_ref, b3_ref, w4_ref, b4_ref, out_ref, y_scr, z_scr, h_scr):
    f32 = jnp.float32
    bf16 = jnp.bfloat16
    xr = x_ref[...].reshape(BT * N, F).astype(bf16)
    y_scr[...] = jnp.dot(xr, wcb_ref[...],
                         preferred_element_type=f32).astype(bf16)
    a = a_ref[...]

    def zstep(g, carry):
        base = g * (ZG * N)
        rhs = jnp.concatenate(
            [y_scr[pl.ds(base + k * N, N), CH:2 * CH] for k in range(ZG)],
            axis=1)
        zz = jnp.dot(a, rhs, preferred_element_type=f32)
        for k in range(ZG):
            z_scr[pl.ds(base + k * N, N), :] = zz[:, k * CH:(k + 1) * CH]
        return carry

    jax.lax.fori_loop(0, BT // ZG, zstep, 0)

    h = y_scr[:, :CH].astype(f32) + z_scr[...] + bch_ref[...]
    h = jnp.where(h > 0, h, jnp.exp(jnp.minimum(h, 0.0)) - 1.0)
    h_scr[...] = h.reshape(BT, N, CH).astype(bf16)

    def d1step(g, acc):
        blk = h_scr[:, pl.ds(g * NG, NG), :].reshape(BT, NG * CH)
        w1blk = w1_ref[pl.ds(g * NG * CH, NG * CH), :]
        return acc + jnp.dot(blk, w1blk, preferred_element_type=f32)

    acc = jax.lax.fori_loop(0, N // NG, d1step,
                            jnp.zeros((BT, H), f32))
    o1 = jnp.maximum(acc + b1_ref[...], 0.0).astype(bf16)
    o2 = jnp.maximum(jnp.dot(o1, w2_ref[...], preferred_element_type=f32)
                     + b2_ref[...], 0.0).astype(bf16)
    o3 = jnp.maximum(jnp.dot(o2, w3_ref[...], preferred_element_type=f32)
                     + b3_ref[...], 0.0).astype(bf16)
    o4 = jnp.dot(o3, w4_ref[...], preferred_element_type=f32) + b4_ref[...]
    out_ref[...] = jax.nn.sigmoid(o4)


def kernel(x, a, W_cheb, b_cheb, W1, b1, W2, b2, W3, b3, W4, b4):
    bf16 = jnp.bfloat16
    wcb = jnp.concatenate([W_cheb[0], W_cheb[1]], axis=1).astype(bf16)
    bch = b_cheb.reshape(1, CH)
    b1r, b2r, b3r, b4r = (b1.reshape(1, -1), b2.reshape(1, -1),
                          b3.reshape(1, -1), b4.reshape(1, -1))
    full = lambda shape: pl.BlockSpec(shape, lambda i: (0,) * len(shape))
    return pl.pallas_call(
        _body,
        grid=(NT,),
        in_specs=[
            pl.BlockSpec((BT, N, F), lambda i: (i, 0, 0)),
            full((N, N)),
            full((F, 2 * CH)),
            full((1, CH)),
            full((N * CH, H)),
            full((1, H)),
            full((H, H // 2)),
            full((1, H // 2)),
            full((H // 2, H // 4)),
            full((1, H // 4)),
            full((H // 4, 1)),
            full((1, 1)),
        ],
        out_specs=pl.BlockSpec((BT, 1), lambda i: (i, 0)),
        out_shape=jax.ShapeDtypeStruct((B, 1), jnp.float32),
        scratch_shapes=[
            pltpu.VMEM((BT * N, 2 * CH), bf16),
            pltpu.VMEM((BT * N, CH), jnp.float32),
            pltpu.VMEM((BT, N, CH), bf16),
        ],
    )(x, a.astype(bf16), wcb, bch, W1.astype(bf16), b1r,
      W2.astype(bf16), b2r, W3.astype(bf16), b3r, W4.astype(bf16), b4r)
